# baseline (speedup 1.0000x reference)
import functools

import jax
import jax.numpy as jnp
from jax import lax
from jax.experimental import pallas as pl
from jax.experimental.pallas import tpu as pltpu
from jax.experimental.pallas import tpu_sc as plsc

B = 4
S = 2048
VOCAB = 100000
MAX_POS = 2048
H = 1024

NC = 2
NS = 16
NW = NC * NS
N_TOK = B * S
SCH = 4
LANES = 16
HT = H // 128
GPC = SCH * HT * B
S_PER_W = S // NW
NCHUNK = S_PER_W // SCH

_mesh = plsc.VectorSubcoreMesh(core_axis_name="c", subcore_axis_name="s")


@functools.partial(
    pl.kernel,
    mesh=_mesh,
    out_type=jax.ShapeDtypeStruct((S * HT * B, 128), jnp.float32),
    scratch_types=[
        pltpu.VMEM((S_PER_W * HT * B,), jnp.int32),
        pltpu.VMEM((S_PER_W * HT * B,), jnp.int32),
        pltpu.VMEM((3, GPC, 128), jnp.float32),
        pltpu.VMEM((2, GPC, 128), jnp.float32),
        pltpu.SemaphoreType.DMA,
        pltpu.SemaphoreType.DMA,
        pltpu.SemaphoreType.DMA,
        pltpu.SemaphoreType.DMA,
        pltpu.SemaphoreType.DMA,
        pltpu.SemaphoreType.DMA,
    ],
)
def _emb_lookup(wtab, ptab, wids, pids, out, widx_v, pidx_v, wbuf, pbuf,
                wsem0, wsem1, psem0, psem1, osem0, osem1):
    wsem = (wsem0, wsem1)
    psem = (psem0, psem1)
    osem = (osem0, osem1)
    w = lax.axis_index("s") * NC + lax.axis_index("c")
    idx_per_w = S_PER_W * HT * B
    base = w * idx_per_w

    pltpu.sync_copy(wids.at[pl.ds(base, idx_per_w)], widx_v)
    pltpu.sync_copy(pids.at[pl.ds(base, idx_per_w)], pidx_v)

    def issue_gather(i):
        return (
            pltpu.async_copy(
                wtab.at[widx_v.at[pl.ds(i * GPC, GPC)]], wbuf.at[i % 3],
                wsem[i % 2]),
            pltpu.async_copy(
                ptab.at[pidx_v.at[pl.ds(i * GPC, GPC)]], pbuf.at[i % 2],
                psem[i % 2]),
        )

    pending_g = {0: issue_gather(0)}
    pending_o = {}

    for i in range(NCHUNK):
        ws, ps = i % 3, i % 2
        if i + 1 < NCHUNK:
            if i - 2 in pending_o:
                pending_o.pop(i - 2).wait()
            pending_g[i + 1] = issue_gather(i + 1)
        wc, pc = pending_g.pop(i)
        wc.wait()
        pc.wait()

        def body(j, _, _ws=ws, _ps=ps):
            for k in range(8):
                x = pbuf[_ps, j, pl.ds(k * LANES, LANES)]
                plsc.addupdate(wbuf.at[_ws, j, pl.ds(k * LANES, LANES)], x)
            return 0

        lax.fori_loop(0, GPC, body, 0, unroll=False)

        pending_o[i] = pltpu.async_copy(
            wbuf.at[ws], out.at[pl.ds((base + i * GPC), GPC)], osem[i % 2])

    for i in sorted(pending_o):
        pending_o[i].wait()


def kernel(input_ids, position_ids, word_embeddings, position_embeddings):
    ht_arange = jnp.arange(HT, dtype=jnp.int32)[None, :, None]
    ids_t = jnp.swapaxes(input_ids, 0, 1)
    pos_t = jnp.swapaxes(position_ids, 0, 1)
    wids = (ids_t[:, None, :] * HT + ht_arange).reshape(-1)
    pids = (pos_t[:, None, :] * HT + ht_arange).reshape(-1)
    wtab = word_embeddings.reshape(VOCAB * HT, 128)
    ptab = position_embeddings.reshape(MAX_POS * HT, 128)
    out2 = _emb_lookup(wtab, ptab, wids, pids)
    out4 = out2.reshape(S, HT, B, 128)
    return jnp.transpose(out4, (0, 2, 1, 3)).reshape(S, B, H)


# speedup vs baseline: 1.3583x; 1.3583x over previous
import functools

import jax
import jax.numpy as jnp
from jax import lax
from jax.experimental import pallas as pl
from jax.experimental.pallas import tpu as pltpu
from jax.experimental.pallas import tpu_sc as plsc

B = 4
S = 2048
VOCAB = 100000
MAX_POS = 2048
H = 1024

NC = 2
NS = 16
NW = NC * NS
LANES = 16
HT = H // 128
SBLK = NW // B
S_PER_W = S // SBLK
CHUNK = 16
NCHUNK = S_PER_W // CHUNK

_mesh = plsc.VectorSubcoreMesh(core_axis_name="c", subcore_axis_name="s")


@functools.partial(
    pl.kernel,
    mesh=_mesh,
    out_type=jax.ShapeDtypeStruct((S, HT, B, 128), jnp.float32),
    scratch_types=[
        pltpu.VMEM((S_PER_W,), jnp.int32),
        pltpu.VMEM((S_PER_W,), jnp.int32),
        pltpu.VMEM((3, CHUNK, HT, 128), jnp.float32),
        pltpu.VMEM((2, CHUNK, HT, 128), jnp.float32),
        pltpu.SemaphoreType.DMA,
        pltpu.SemaphoreType.DMA,
        pltpu.SemaphoreType.DMA,
        pltpu.SemaphoreType.DMA,
        pltpu.SemaphoreType.DMA,
        pltpu.SemaphoreType.DMA,
    ],
)
def _emb_lookup(wtab, ptab, wids, pids, out, widx_v, pidx_v, wbuf, pbuf,
                wsem0, wsem1, psem0, psem1, osem0, osem1):
    wsem = (wsem0, wsem1)
    psem = (psem0, psem1)
    osem = (osem0, osem1)
    w = lax.axis_index("s") * NC + lax.axis_index("c")
    b = w % B
    s0 = (w // B) * S_PER_W

    pltpu.sync_copy(wids.at[b, pl.ds(s0, S_PER_W)], widx_v)
    pltpu.sync_copy(pids.at[b, pl.ds(s0, S_PER_W)], pidx_v)

    def issue_gather(i):
        return (
            pltpu.async_copy(
                wtab.at[widx_v.at[pl.ds(i * CHUNK, CHUNK)]], wbuf.at[i % 3],
                wsem[i % 2]),
            pltpu.async_copy(
                ptab.at[pidx_v.at[pl.ds(i * CHUNK, CHUNK)]], pbuf.at[i % 2],
                psem[i % 2]),
        )

    pending_g = {0: issue_gather(0)}
    pending_o = {}

    for i in range(NCHUNK):
        ws, ps = i % 3, i % 2
        if i + 1 < NCHUNK:
            if i - 2 in pending_o:
                pending_o.pop(i - 2).wait()
            pending_g[i + 1] = issue_gather(i + 1)
        wc, pc = pending_g.pop(i)
        wc.wait()
        pc.wait()

        def body(r, _, _ws=ws, _ps=ps):
            for kk in range(HT):
                for l0 in range(0, 128, LANES):
                    x = pbuf[_ps, r, kk, pl.ds(l0, LANES)]
                    plsc.addupdate(wbuf.at[_ws, r, kk, pl.ds(l0, LANES)], x)
            return 0

        lax.fori_loop(0, CHUNK, body, 0, unroll=False)

        pending_o[i] = pltpu.async_copy(
            wbuf.at[ws], out.at[pl.ds(s0 + i * CHUNK, CHUNK), :, b],
            osem[i % 2])

    for i in sorted(pending_o):
        pending_o[i].wait()


def kernel(input_ids, position_ids, word_embeddings, position_embeddings):
    wtab = word_embeddings.reshape(VOCAB, HT, 128)
    ptab = position_embeddings.reshape(MAX_POS, HT, 128)
    out4 = _emb_lookup(wtab, ptab, input_ids, position_ids)
    return jnp.transpose(out4, (0, 2, 1, 3)).reshape(S, B, H)


# speedup vs baseline: 4.1167x; 3.0308x over previous
import functools

import jax
import jax.numpy as jnp
from jax import lax
from jax.experimental import pallas as pl
from jax.experimental.pallas import tpu as pltpu
from jax.experimental.pallas import tpu_sc as plsc

B = 4
S = 2048
VOCAB = 100000
MAX_POS = 2048
H = 1024

NC = 2
NS = 16
NW = NC * NS
N_TOK = B * S
CHUNK = 16
SCH = CHUNK // B
LANES = 16
HT = H // 128
S_PER_W = S // NW
NCHUNK = S_PER_W // SCH

_mesh = plsc.VectorSubcoreMesh(core_axis_name="c", subcore_axis_name="s")


@functools.partial(
    pl.kernel,
    mesh=_mesh,
    out_type=jax.ShapeDtypeStruct((S * HT * B, 128), jnp.float32),
    scratch_types=[
        pltpu.VMEM((S_PER_W * B,), jnp.int32),
        pltpu.VMEM((S_PER_W * B,), jnp.int32),
        pltpu.VMEM((2, CHUNK, H), jnp.float32),
        pltpu.VMEM((2, CHUNK, H), jnp.float32),
        pltpu.VMEM((2, CHUNK * HT, 128), jnp.float32),
        pltpu.SemaphoreType.DMA,
        pltpu.SemaphoreType.DMA,
        pltpu.SemaphoreType.DMA,
        pltpu.SemaphoreType.DMA,
        pltpu.SemaphoreType.DMA,
        pltpu.SemaphoreType.DMA,
    ],
)
def _emb_lookup(wtab, ptab, wids, pids, out, widx_v, pidx_v, wbuf, pbuf, obuf,
                wsem0, wsem1, psem0, psem1, osem0, osem1):
    wsem = (wsem0, wsem1)
    psem = (psem0, psem1)
    osem = (osem0, osem1)
    w = lax.axis_index("s") * NC + lax.axis_index("c")
    tok_per_w = S_PER_W * B
    base = w * tok_per_w

    pltpu.sync_copy(wids.at[pl.ds(base, tok_per_w)], widx_v)
    pltpu.sync_copy(pids.at[pl.ds(base, tok_per_w)], pidx_v)

    def issue_gather(i):
        return (
            pltpu.async_copy(
                wtab.at[widx_v.at[pl.ds(i * CHUNK, CHUNK)]], wbuf.at[i % 2],
                wsem[i % 2]),
            pltpu.async_copy(
                ptab.at[pidx_v.at[pl.ds(i * CHUNK, CHUNK)]], pbuf.at[i % 2],
                psem[i % 2]),
        )

    pending_g = {0: issue_gather(0)}
    pending_o = {}

    for i in range(NCHUNK):
        sl = i % 2
        if i - 2 in pending_o:
            pending_o.pop(i - 2).wait()
        if i + 1 < NCHUNK:
            pending_g[i + 1] = issue_gather(i + 1)
        wc, pc = pending_g.pop(i)
        wc.wait()
        pc.wait()

        def body(r, _, _sl=sl):
            si = r // B
            b = r % B
            orow0 = si * (HT * B) + b
            wrow = wbuf.at[_sl, r]
            prow = pbuf.at[_sl, r]
            for kk in range(HT):
                opiece = obuf.at[_sl, orow0 + kk * B]
                for l0 in range(0, 128, LANES):
                    x = wrow[pl.ds(kk * 128 + l0, LANES)]
                    y = prow[pl.ds(kk * 128 + l0, LANES)]
                    opiece[pl.ds(l0, LANES)] = x + y
            return 0

        lax.fori_loop(0, CHUNK, body, 0, unroll=False)

        pending_o[i] = pltpu.async_copy(
            obuf.at[sl], out.at[pl.ds((base + i * CHUNK) * HT, CHUNK * HT)],
            osem[sl])

    for i in sorted(pending_o):
        pending_o[i].wait()


def kernel(input_ids, position_ids, word_embeddings, position_embeddings):
    wids = jnp.swapaxes(input_ids, 0, 1).reshape(-1)
    pids = jnp.swapaxes(position_ids, 0, 1).reshape(-1)
    out2 = _emb_lookup(word_embeddings, position_embeddings, wids, pids)
    out4 = out2.reshape(S, HT, B, 128)
    return jnp.transpose(out4, (0, 2, 1, 3)).reshape(S, B, H)
